# baseline (speedup 1.0000x reference)
import functools

import jax
import jax.numpy as jnp
from jax import lax
from jax.experimental import pallas as pl
from jax.experimental.pallas import tpu as pltpu
from jax.experimental.pallas import tpu_sc as plsc

EPS = 1e-5

_NC = 2
_NS = 16
_NW = _NC * _NS
_LANES = 16

_C0 = 120
_C1 = 80

_G = 16
_RUNROLL = 8


@functools.lru_cache(maxsize=None)
def _make_bow(B, L, H):
    assert B % _NW == 0
    spw = B // _NW
    assert L == _C0 + _C1
    HW = H // 2
    nch = HW // _LANES
    ngroups = spw // _G
    nsup = ngroups // 2
    npairs = _G // 2
    assert ngroups % 2 == 0 and L % _RUNROLL == 0

    mesh = plsc.VectorSubcoreMesh(
        core_axis_name="c", subcore_axis_name="s", num_cores=_NC,
        num_subcores=_NS)

    @functools.partial(
        pl.kernel,
        out_type=jax.ShapeDtypeStruct((B * H,), jnp.float32),
        mesh=mesh,
        scratch_types=[
            pltpu.VMEM((_G * L,), jnp.int32),
            pltpu.VMEM((_G * L,), jnp.int32),
            pltpu.VMEM((L, HW), jnp.float32),
            pltpu.VMEM((L, HW), jnp.float32),
            pltpu.VMEM((_G * H,), jnp.float32),
            pltpu.SemaphoreType.DMA,
            pltpu.SemaphoreType.DMA,
            pltpu.SemaphoreType.DMA,
            pltpu.SemaphoreType.DMA,
        ],
        compiler_params=pltpu.CompilerParams(use_tc_tiling_on_sc=False),
    )
    def bow_kernel(x_hbm, table_hbm, out_hbm, idx0, idx1, rows0, rows1,
                   outg, isem0, isem1, rsem0, rsem1):
        wid = lax.axis_index("s") * _NC + lax.axis_index("c")
        base = wid * spw
        scale = jnp.float32(1.0 / L)

        def issue_idx(g, ibuf, isem):
            pltpu.async_copy(
                x_hbm.at[pl.ds((base + g * _G) * L, _G * L)], ibuf, isem)

        def wait_idx(ibuf, isem):
            pltpu.make_async_copy(
                x_hbm.at[pl.ds(0, _G * L)], ibuf, isem).wait()

        def issue_gather(ibuf, j, rbuf, rsem):
            pltpu.async_copy(
                table_hbm.at[ibuf.at[pl.ds(j * L, _C0)]],
                rbuf.at[pl.ds(0, _C0)], rsem)
            pltpu.async_copy(
                table_hbm.at[ibuf.at[pl.ds(j * L + _C0, _C1)]],
                rbuf.at[pl.ds(_C0, _C1)], rsem)

        def wait_gather(ibuf, rbuf, rsem):
            pltpu.make_async_copy(
                table_hbm.at[ibuf.at[pl.ds(0, _C0)]],
                rbuf.at[pl.ds(0, _C0)], rsem).wait()
            pltpu.make_async_copy(
                table_hbm.at[ibuf.at[pl.ds(0, _C1)]],
                rbuf.at[pl.ds(_C0, _C1)], rsem).wait()

        def reduce_store(rbuf, j):
            def red(r, acc):
                out = list(acc)
                for c in range(nch):
                    lo = []
                    hi = []
                    for u in range(_RUNROLL):
                        r0 = r * _RUNROLL + u
                        w = lax.bitcast_convert_type(
                            rbuf[r0, pl.ds(c * _LANES, _LANES)], jnp.uint32)
                        a = lax.bitcast_convert_type(w << 16, jnp.float32)
                        b = lax.bitcast_convert_type(
                            w & jnp.uint32(0xFFFF0000), jnp.float32)
                        lo.append(a)
                        hi.append(b)
                    slo = ((lo[0] + lo[1]) + (lo[2] + lo[3]) +
                           ((lo[4] + lo[5]) + (lo[6] + lo[7])))
                    shi = ((hi[0] + hi[1]) + (hi[2] + hi[3]) +
                           ((hi[4] + hi[5]) + (hi[6] + hi[7])))
                    out[c] = out[c] + slo
                    out[nch + c] = out[nch + c] + shi
                return tuple(out)

            zero = jnp.zeros((_LANES,), jnp.float32)
            acc = lax.fori_loop(0, L // _RUNROLL, red, (zero,) * (2 * nch))
            for c in range(nch):
                outg[pl.ds(j * H + c * _LANES, _LANES)] = acc[c] * scale
            for c in range(nch):
                outg[pl.ds(j * H + (nch + c) * _LANES, _LANES)] = (
                    acc[nch + c] * scale)

        def process_group(ibuf, g):
            issue_gather(ibuf, 0, rows0, rsem0)

            def pair_body(p, carry):
                j0 = 2 * p
                issue_gather(ibuf, j0 + 1, rows1, rsem1)
                wait_gather(ibuf, rows0, rsem0)
                reduce_store(rows0, j0)

                @pl.when(p < npairs - 1)
                def _():
                    issue_gather(ibuf, j0 + 2, rows0, rsem0)

                wait_gather(ibuf, rows1, rsem1)
                reduce_store(rows1, j0 + 1)
                return carry

            lax.fori_loop(0, npairs, pair_body, 0)
            pltpu.sync_copy(
                outg, out_hbm.at[pl.ds((base + g * _G) * H, _G * H)])

        issue_idx(0, idx0, isem0)

        def sup_body(k, carry):
            g0 = 2 * k
            issue_idx(g0 + 1, idx1, isem1)
            wait_idx(idx0, isem0)
            process_group(idx0, g0)

            @pl.when(k < nsup - 1)
            def _():
                issue_idx(g0 + 2, idx0, isem0)

            wait_idx(idx1, isem1)
            process_group(idx1, g0 + 1)
            return carry

        lax.fori_loop(0, nsup, sup_body, 0)

    return bow_kernel


def _mlp_body(bow_ref, w1t_ref, b1_ref, gamma_ref, beta_ref, w2t_ref, b2_ref,
              out_ref):
    bow = bow_ref[...]
    h = jnp.dot(bow, w1t_ref[...], preferred_element_type=jnp.float32)
    h = h + b1_ref[...]
    mu = jnp.mean(h, axis=0, keepdims=True)
    d = h - mu
    var = jnp.mean(d * d, axis=0, keepdims=True)
    hn = d * lax.rsqrt(var + EPS) * gamma_ref[...] + beta_ref[...]
    h2 = jnp.maximum(hn, 0.0)
    out = jnp.dot(h2, w2t_ref[...], preferred_element_type=jnp.float32)
    out_ref[...] = out + b2_ref[...]


def kernel(x, table, W1, b1, gamma, beta, W2, b2):
    B, L = x.shape
    _, H = table.shape
    O = W2.shape[0]
    HW = H // 2
    x_flat = x.reshape(-1).astype(jnp.int32)
    V = table.shape[0]
    tb = table.astype(jnp.bfloat16)
    packed = lax.bitcast_convert_type(tb.reshape(V, HW, 2), jnp.float32)
    bow = _make_bow(B, L, H)(x_flat, packed).reshape(B, H)
    perm = jnp.concatenate([jnp.arange(0, H, 2), jnp.arange(1, H, 2)])
    out = pl.pallas_call(
        _mlp_body,
        out_shape=jax.ShapeDtypeStruct((B, O), jnp.float32),
    )(bow, W1.T[perm, :], b1.reshape(1, H), gamma.reshape(1, H),
      beta.reshape(1, H), W2.T, b2.reshape(1, O))
    return out


# speedup vs baseline: 1.5615x; 1.5615x over previous
import functools

import jax
import jax.numpy as jnp
from jax import lax
from jax.experimental import pallas as pl
from jax.experimental.pallas import tpu as pltpu
from jax.experimental.pallas import tpu_sc as plsc

EPS = 1e-5

_NC = 2
_NS = 16
_NW = _NC * _NS
_LANES = 16

_C0 = 120
_C1 = 80

_G = 16
_RUNROLL = 8


@functools.lru_cache(maxsize=None)
def _make_bow(B, L, H):
    assert B % _NW == 0
    spw = B // _NW
    assert L == _C0 + _C1
    HW = H // 2
    nch = HW // _LANES
    ngroups = spw // _G
    nsup = ngroups // 2
    npairs = _G // 2
    assert ngroups % 2 == 0 and L % _RUNROLL == 0

    mesh = plsc.VectorSubcoreMesh(
        core_axis_name="c", subcore_axis_name="s", num_cores=_NC,
        num_subcores=_NS)

    @functools.partial(
        pl.kernel,
        out_type=jax.ShapeDtypeStruct((B * H,), jnp.float32),
        mesh=mesh,
        scratch_types=[
            pltpu.VMEM((_G * L,), jnp.int32),
            pltpu.VMEM((_G * L,), jnp.int32),
            pltpu.VMEM((L, HW), jnp.float32),
            pltpu.VMEM((L, HW), jnp.float32),
            pltpu.VMEM((_G * H,), jnp.float32),
            pltpu.SemaphoreType.DMA,
            pltpu.SemaphoreType.DMA,
            pltpu.SemaphoreType.DMA,
            pltpu.SemaphoreType.DMA,
        ],
        compiler_params=pltpu.CompilerParams(use_tc_tiling_on_sc=False),
    )
    def bow_kernel(x_hbm, table_hbm, out_hbm, idx0, idx1, rows0, rows1,
                   outg, isem0, isem1, rsem0, rsem1):
        wid = lax.axis_index("s") * _NC + lax.axis_index("c")
        base = wid * spw
        scale = jnp.float32(1.0 / L)

        def issue_idx(g, ibuf, isem):
            pltpu.async_copy(
                x_hbm.at[pl.ds((base + g * _G) * L, _G * L)], ibuf, isem)

        def wait_idx(ibuf, isem):
            pltpu.make_async_copy(
                x_hbm.at[pl.ds(0, _G * L)], ibuf, isem).wait()

        def issue_gather(ibuf, j, rbuf, rsem):
            pltpu.async_copy(
                table_hbm.at[ibuf.at[pl.ds(j * L, _C0)]],
                rbuf.at[pl.ds(0, _C0)], rsem)
            pltpu.async_copy(
                table_hbm.at[ibuf.at[pl.ds(j * L + _C0, _C1)]],
                rbuf.at[pl.ds(_C0, _C1)], rsem)

        def wait_gather(ibuf, rbuf, rsem):
            pltpu.make_async_copy(
                table_hbm.at[ibuf.at[pl.ds(0, _C0)]],
                rbuf.at[pl.ds(0, _C0)], rsem).wait()
            pltpu.make_async_copy(
                table_hbm.at[ibuf.at[pl.ds(0, _C1)]],
                rbuf.at[pl.ds(_C0, _C1)], rsem).wait()

        def reduce_store(rbuf, j):
            def red(r, acc):
                out = list(acc)
                for c in range(nch):
                    lo = []
                    hi = []
                    for u in range(_RUNROLL):
                        r0 = r * _RUNROLL + u
                        w = lax.bitcast_convert_type(
                            rbuf[r0, pl.ds(c * _LANES, _LANES)], jnp.uint32)
                        a = lax.bitcast_convert_type(w << 16, jnp.float32)
                        b = lax.bitcast_convert_type(
                            w & jnp.uint32(0xFFFF0000), jnp.float32)
                        lo.append(a)
                        hi.append(b)
                    slo = ((lo[0] + lo[1]) + (lo[2] + lo[3]) +
                           ((lo[4] + lo[5]) + (lo[6] + lo[7])))
                    shi = ((hi[0] + hi[1]) + (hi[2] + hi[3]) +
                           ((hi[4] + hi[5]) + (hi[6] + hi[7])))
                    out[c] = out[c] + slo
                    out[nch + c] = out[nch + c] + shi
                return tuple(out)

            zero = jnp.zeros((_LANES,), jnp.float32)
            acc = lax.fori_loop(0, L // _RUNROLL, red, (zero,) * (2 * nch))
            for c in range(nch):
                outg[pl.ds(j * H + c * _LANES, _LANES)] = acc[c] * scale
            for c in range(nch):
                outg[pl.ds(j * H + (nch + c) * _LANES, _LANES)] = (
                    acc[nch + c] * scale)

        def process_group(ibuf, g):
            issue_gather(ibuf, 0, rows0, rsem0)

            def pair_body(p, carry):
                j0 = 2 * p
                issue_gather(ibuf, j0 + 1, rows1, rsem1)
                wait_gather(ibuf, rows0, rsem0)
                reduce_store(rows0, j0)

                @pl.when(p < npairs - 1)
                def _():
                    issue_gather(ibuf, j0 + 2, rows0, rsem0)

                wait_gather(ibuf, rows1, rsem1)
                reduce_store(rows1, j0 + 1)
                return carry

            lax.fori_loop(0, npairs, pair_body, 0)
            pltpu.sync_copy(
                outg, out_hbm.at[pl.ds((base + g * _G) * H, _G * H)])

        issue_idx(0, idx0, isem0)

        def sup_body(k, carry):
            g0 = 2 * k
            issue_idx(g0 + 1, idx1, isem1)
            wait_idx(idx0, isem0)
            process_group(idx0, g0)

            @pl.when(k < nsup - 1)
            def _():
                issue_idx(g0 + 2, idx0, isem0)

            wait_idx(idx1, isem1)
            process_group(idx1, g0 + 1)
            return carry

        lax.fori_loop(0, nsup, sup_body, 0)

    return bow_kernel


def _mlp_body(bow_ref, w1t_ref, b1_ref, gamma_ref, beta_ref, w2t_ref, b2_ref,
              out_ref):
    bow = bow_ref[...]
    h = jnp.dot(bow, w1t_ref[...], preferred_element_type=jnp.float32)
    h = h + b1_ref[...]
    mu = jnp.mean(h, axis=0, keepdims=True)
    d = h - mu
    var = jnp.mean(d * d, axis=0, keepdims=True)
    hn = d * lax.rsqrt(var + EPS) * gamma_ref[...] + beta_ref[...]
    h2 = jnp.maximum(hn, 0.0)
    out = jnp.dot(h2, w2t_ref[...], preferred_element_type=jnp.float32)
    out_ref[...] = out + b2_ref[...]


def kernel(x, table, W1, b1, gamma, beta, W2, b2):
    B, L = x.shape
    _, H = table.shape
    O = W2.shape[0]
    HW = H // 2
    x_flat = x.reshape(-1).astype(jnp.int32)
    lo = lax.bitcast_convert_type(
        table[:, :HW].astype(jnp.bfloat16), jnp.uint16).astype(jnp.uint32)
    hi = lax.bitcast_convert_type(
        table[:, HW:].astype(jnp.bfloat16), jnp.uint16).astype(jnp.uint32)
    packed = lax.bitcast_convert_type(lo | (hi << 16), jnp.float32)
    packed = lax.optimization_barrier(packed)
    bow = _make_bow(B, L, H)(x_flat, packed).reshape(B, H)
    out = pl.pallas_call(
        _mlp_body,
        out_shape=jax.ShapeDtypeStruct((B, O), jnp.float32),
    )(bow, W1.T, b1.reshape(1, H), gamma.reshape(1, H), beta.reshape(1, H),
      W2.T, b2.reshape(1, O))
    return out


# speedup vs baseline: 1.8253x; 1.1689x over previous
import functools

import jax
import jax.numpy as jnp
from jax import lax
from jax.experimental import pallas as pl
from jax.experimental.pallas import tpu as pltpu
from jax.experimental.pallas import tpu_sc as plsc

EPS = 1e-5

_NC = 2
_NS = 16
_NW = _NC * _NS
_LANES = 16

_C0 = 120
_C1 = 80

_G = 16
_RUNROLL = 8


@functools.lru_cache(maxsize=None)
def _make_bow(B, L, H):
    assert B % _NW == 0
    spw = B // _NW
    assert L == _C0 + _C1
    HW = H // 2
    nch = HW // _LANES
    ngroups = spw // _G
    nsup = ngroups // 2
    npairs = _G // 2
    assert ngroups % 2 == 0 and L % _RUNROLL == 0

    mesh = plsc.VectorSubcoreMesh(
        core_axis_name="c", subcore_axis_name="s", num_cores=_NC,
        num_subcores=_NS)

    @functools.partial(
        pl.kernel,
        out_type=jax.ShapeDtypeStruct((B * H,), jnp.float32),
        mesh=mesh,
        scratch_types=[
            pltpu.VMEM((_G * L,), jnp.int32),
            pltpu.VMEM((_G * L,), jnp.int32),
            pltpu.VMEM((L, HW), jnp.float32),
            pltpu.VMEM((L, HW), jnp.float32),
            pltpu.VMEM((_G * H,), jnp.float32),
            pltpu.SemaphoreType.DMA,
            pltpu.SemaphoreType.DMA,
            pltpu.SemaphoreType.DMA,
            pltpu.SemaphoreType.DMA,
        ],
        compiler_params=pltpu.CompilerParams(use_tc_tiling_on_sc=False),
    )
    def bow_kernel(x_hbm, table_hbm, out_hbm, idx0, idx1, rows0, rows1,
                   outg, isem0, isem1, rsem0, rsem1):
        wid = lax.axis_index("s") * _NC + lax.axis_index("c")
        base = wid * spw
        scale = jnp.float32(1.0 / L)

        def issue_idx(g, ibuf, isem):
            pltpu.async_copy(
                x_hbm.at[pl.ds((base + g * _G) * L, _G * L)], ibuf, isem)

        def wait_idx(ibuf, isem):
            pltpu.make_async_copy(
                x_hbm.at[pl.ds(0, _G * L)], ibuf, isem).wait()

        def issue_gather(ibuf, j, rbuf, rsem):
            pltpu.async_copy(
                table_hbm.at[ibuf.at[pl.ds(j * L, _C0)]],
                rbuf.at[pl.ds(0, _C0)], rsem)
            pltpu.async_copy(
                table_hbm.at[ibuf.at[pl.ds(j * L + _C0, _C1)]],
                rbuf.at[pl.ds(_C0, _C1)], rsem)

        def wait_gather(ibuf, rbuf, rsem):
            pltpu.make_async_copy(
                table_hbm.at[ibuf.at[pl.ds(0, _C0)]],
                rbuf.at[pl.ds(0, _C0)], rsem).wait()
            pltpu.make_async_copy(
                table_hbm.at[ibuf.at[pl.ds(0, _C1)]],
                rbuf.at[pl.ds(_C0, _C1)], rsem).wait()

        def reduce_store(rbuf, j):
            def red(r, acc):
                out = list(acc)
                for c in range(nch):
                    lo = []
                    hi = []
                    for u in range(_RUNROLL):
                        r0 = r * _RUNROLL + u
                        w = lax.bitcast_convert_type(
                            rbuf[r0, pl.ds(c * _LANES, _LANES)], jnp.uint32)
                        a = lax.bitcast_convert_type(w << 16, jnp.float32)
                        b = lax.bitcast_convert_type(
                            w & jnp.uint32(0xFFFF0000), jnp.float32)
                        lo.append(a)
                        hi.append(b)
                    slo = ((lo[0] + lo[1]) + (lo[2] + lo[3]) +
                           ((lo[4] + lo[5]) + (lo[6] + lo[7])))
                    shi = ((hi[0] + hi[1]) + (hi[2] + hi[3]) +
                           ((hi[4] + hi[5]) + (hi[6] + hi[7])))
                    out[c] = out[c] + slo
                    out[nch + c] = out[nch + c] + shi
                return tuple(out)

            zero = jnp.zeros((_LANES,), jnp.float32)
            acc = lax.fori_loop(0, L // _RUNROLL, red, (zero,) * (2 * nch))
            for c in range(nch):
                outg[pl.ds(j * H + c * _LANES, _LANES)] = acc[c] * scale
            for c in range(nch):
                outg[pl.ds(j * H + (nch + c) * _LANES, _LANES)] = (
                    acc[nch + c] * scale)

        def process_group(ibuf, g):
            issue_gather(ibuf, 0, rows0, rsem0)

            def pair_body(p, carry):
                j0 = 2 * p
                issue_gather(ibuf, j0 + 1, rows1, rsem1)
                wait_gather(ibuf, rows0, rsem0)
                reduce_store(rows0, j0)

                @pl.when(p < npairs - 1)
                def _():
                    issue_gather(ibuf, j0 + 2, rows0, rsem0)

                wait_gather(ibuf, rows1, rsem1)
                reduce_store(rows1, j0 + 1)
                return carry

            lax.fori_loop(0, npairs, pair_body, 0)
            pltpu.sync_copy(
                outg, out_hbm.at[pl.ds((base + g * _G) * H, _G * H)])

        issue_idx(0, idx0, isem0)

        def sup_body(k, carry):
            g0 = 2 * k
            issue_idx(g0 + 1, idx1, isem1)
            wait_idx(idx0, isem0)
            process_group(idx0, g0)

            @pl.when(k < nsup - 1)
            def _():
                issue_idx(g0 + 2, idx0, isem0)

            wait_idx(idx1, isem1)
            process_group(idx1, g0 + 1)
            return carry

        lax.fori_loop(0, nsup, sup_body, 0)

    return bow_kernel


def _pack_body(lo_ref, hi_ref, out_ref):
    lou = lax.bitcast_convert_type(
        lo_ref[...].astype(jnp.bfloat16), jnp.uint16).astype(jnp.uint32)
    hiu = lax.bitcast_convert_type(
        hi_ref[...].astype(jnp.bfloat16), jnp.uint16).astype(jnp.uint32)
    out_ref[...] = lax.bitcast_convert_type(lou | (hiu << 16), jnp.float32)


def _mlp_body(bow_ref, w1t_ref, b1_ref, gamma_ref, beta_ref, w2t_ref, b2_ref,
              out_ref):
    bow = bow_ref[...]
    h = jnp.dot(bow, w1t_ref[...], preferred_element_type=jnp.float32)
    h = h + b1_ref[...]
    mu = jnp.mean(h, axis=0, keepdims=True)
    d = h - mu
    var = jnp.mean(d * d, axis=0, keepdims=True)
    hn = d * lax.rsqrt(var + EPS) * gamma_ref[...] + beta_ref[...]
    h2 = jnp.maximum(hn, 0.0)
    out = jnp.dot(h2, w2t_ref[...], preferred_element_type=jnp.float32)
    out_ref[...] = out + b2_ref[...]


def kernel(x, table, W1, b1, gamma, beta, W2, b2):
    B, L = x.shape
    _, H = table.shape
    O = W2.shape[0]
    HW = H // 2
    x_flat = x.reshape(-1).astype(jnp.int32)
    V = table.shape[0]
    tabT = table.T
    CH = 40960
    packedT = pl.pallas_call(
        _pack_body,
        grid=(pl.cdiv(V, CH),),
        in_specs=[
            pl.BlockSpec((HW, CH), lambda i: (0, i)),
            pl.BlockSpec((HW, CH), lambda i: (1, i)),
        ],
        out_specs=pl.BlockSpec((HW, CH), lambda i: (0, i)),
        out_shape=jax.ShapeDtypeStruct((HW, V), jnp.float32),
    )(tabT, tabT)
    bow = _make_bow(B, L, H)(x_flat, packedT.T).reshape(B, H)
    out = pl.pallas_call(
        _mlp_body,
        out_shape=jax.ShapeDtypeStruct((B, O), jnp.float32),
    )(bow, W1.T, b1.reshape(1, H), gamma.reshape(1, H), beta.reshape(1, H),
      W2.T, b2.reshape(1, O))
    return out


# speedup vs baseline: 1.8559x; 1.0168x over previous
import functools

import jax
import jax.numpy as jnp
from jax import lax
from jax.experimental import pallas as pl
from jax.experimental.pallas import tpu as pltpu
from jax.experimental.pallas import tpu_sc as plsc

EPS = 1e-5

_NC = 2
_NS = 16
_NW = _NC * _NS
_LANES = 16

_C0 = 120
_C1 = 80

_G = 16
_RUNROLL = 8


@functools.lru_cache(maxsize=None)
def _make_bow(B, L, H):
    assert B % _NW == 0
    spw = B // _NW
    assert L == _C0 + _C1
    nch = H // _LANES
    ngroups = spw // _G
    nsup = ngroups // 2
    npairs = _G // 2
    assert ngroups % 2 == 0 and L % _RUNROLL == 0

    mesh = plsc.VectorSubcoreMesh(
        core_axis_name="c", subcore_axis_name="s", num_cores=_NC,
        num_subcores=_NS)

    @functools.partial(
        pl.kernel,
        out_type=jax.ShapeDtypeStruct((B * H,), jnp.float32),
        mesh=mesh,
        scratch_types=[
            pltpu.VMEM((_G * L,), jnp.int32),
            pltpu.VMEM((_G * L,), jnp.int32),
            pltpu.VMEM((L, H), jnp.float32),
            pltpu.VMEM((L, H), jnp.float32),
            pltpu.VMEM((_G * H,), jnp.float32),
            pltpu.SemaphoreType.DMA,
            pltpu.SemaphoreType.DMA,
            pltpu.SemaphoreType.DMA,
            pltpu.SemaphoreType.DMA,
        ],
        compiler_params=pltpu.CompilerParams(use_tc_tiling_on_sc=False),
    )
    def bow_kernel(x_hbm, table_hbm, out_hbm, idx0, idx1, rows0, rows1,
                   outg, isem0, isem1, rsem0, rsem1):
        wid = lax.axis_index("s") * _NC + lax.axis_index("c")
        base = wid * spw
        scale = jnp.float32(1.0 / L)

        def issue_idx(g, ibuf, isem):
            pltpu.async_copy(
                x_hbm.at[pl.ds((base + g * _G) * L, _G * L)], ibuf, isem)

        def wait_idx(ibuf, isem):
            pltpu.make_async_copy(
                x_hbm.at[pl.ds(0, _G * L)], ibuf, isem).wait()

        def issue_gather(ibuf, j, rbuf, rsem):
            pltpu.async_copy(
                table_hbm.at[ibuf.at[pl.ds(j * L, _C0)]],
                rbuf.at[pl.ds(0, _C0)], rsem)
            pltpu.async_copy(
                table_hbm.at[ibuf.at[pl.ds(j * L + _C0, _C1)]],
                rbuf.at[pl.ds(_C0, _C1)], rsem)

        def wait_gather(ibuf, rbuf, rsem):
            pltpu.make_async_copy(
                table_hbm.at[ibuf.at[pl.ds(0, _C0)]],
                rbuf.at[pl.ds(0, _C0)], rsem).wait()
            pltpu.make_async_copy(
                table_hbm.at[ibuf.at[pl.ds(0, _C1)]],
                rbuf.at[pl.ds(_C0, _C1)], rsem).wait()

        def reduce_store(rbuf, j):
            def red(r, acc):
                out = list(acc)
                for c in range(nch):
                    t = []
                    for u in range(0, _RUNROLL, 2):
                        r0 = r * _RUNROLL + u
                        t.append(rbuf[r0, pl.ds(c * _LANES, _LANES)] +
                                 rbuf[r0 + 1, pl.ds(c * _LANES, _LANES)])
                    s = (t[0] + t[1]) + (t[2] + t[3])
                    out[c] = out[c] + s
                return tuple(out)

            zero = jnp.zeros((_LANES,), jnp.float32)
            acc = lax.fori_loop(0, L // _RUNROLL, red, (zero,) * nch)
            for c in range(nch):
                outg[pl.ds(j * H + c * _LANES, _LANES)] = acc[c] * scale

        def process_group(ibuf, g):
            issue_gather(ibuf, 0, rows0, rsem0)

            def pair_body(p, carry):
                j0 = 2 * p
                issue_gather(ibuf, j0 + 1, rows1, rsem1)
                wait_gather(ibuf, rows0, rsem0)
                reduce_store(rows0, j0)

                @pl.when(p < npairs - 1)
                def _():
                    issue_gather(ibuf, j0 + 2, rows0, rsem0)

                wait_gather(ibuf, rows1, rsem1)
                reduce_store(rows1, j0 + 1)
                return carry

            lax.fori_loop(0, npairs, pair_body, 0)
            pltpu.sync_copy(
                outg, out_hbm.at[pl.ds((base + g * _G) * H, _G * H)])

        issue_idx(0, idx0, isem0)

        def sup_body(k, carry):
            g0 = 2 * k
            issue_idx(g0 + 1, idx1, isem1)
            wait_idx(idx0, isem0)
            process_group(idx0, g0)

            @pl.when(k < nsup - 1)
            def _():
                issue_idx(g0 + 2, idx0, isem0)

            wait_idx(idx1, isem1)
            process_group(idx1, g0 + 1)
            return carry

        lax.fori_loop(0, nsup, sup_body, 0)

    return bow_kernel


def _mlp_body(bow_ref, w1t_ref, b1_ref, gamma_ref, beta_ref, w2t_ref, b2_ref,
              out_ref):
    bow = bow_ref[...]
    h = jnp.dot(bow, w1t_ref[...], preferred_element_type=jnp.float32)
    h = h + b1_ref[...]
    mu = jnp.mean(h, axis=0, keepdims=True)
    d = h - mu
    var = jnp.mean(d * d, axis=0, keepdims=True)
    hn = d * lax.rsqrt(var + EPS) * gamma_ref[...] + beta_ref[...]
    h2 = jnp.maximum(hn, 0.0)
    out = jnp.dot(h2, w2t_ref[...], preferred_element_type=jnp.float32)
    out_ref[...] = out + b2_ref[...]


def kernel(x, table, W1, b1, gamma, beta, W2, b2):
    B, L = x.shape
    _, H = table.shape
    O = W2.shape[0]
    x_flat = x.reshape(-1).astype(jnp.int32)
    bow = _make_bow(B, L, H)(x_flat, table).reshape(B, H)
    out = pl.pallas_call(
        _mlp_body,
        out_shape=jax.ShapeDtypeStruct((B, O), jnp.float32),
    )(bow, W1.T, b1.reshape(1, H), gamma.reshape(1, H), beta.reshape(1, H),
      W2.T, b2.reshape(1, O))
    return out
